# Initial kernel scaffold; baseline (speedup 1.0000x reference)
import jax
import jax.numpy as jnp
from jax.experimental import pallas as pl


def kernel(x, router_logits, per_expert_scale, w_gate, w_up, w_down):
    raise NotImplementedError("write your pallas kernel here")


# speedup vs baseline: 2.5273x; 2.5273x over previous
import functools

import jax
import jax.numpy as jnp
from jax import lax
from jax.experimental import pallas as pl
from jax.experimental.pallas import tpu as pltpu
from jax.experimental.pallas import tpu_sc as plsc

T, D, E, F, K = 2048, 1024, 8, 2048, 2
A = T * K
B = 256
G_MAX = 24
NPAD = G_MAX * B

NC, NS, L = 2, 16, 16
NW = NC * NS
TW = T // NW
CH = 32


def _routing_body(logits_ref, scale_ref, ids_ref, w0_ref, w1_ref):
    lg = logits_ref[...]
    iota = lax.broadcasted_iota(jnp.int32, (T, E), 1)
    big = jnp.int32(E)
    m1 = jnp.max(lg, axis=1, keepdims=True)
    a1 = jnp.min(jnp.where(lg == m1, iota, big), axis=1, keepdims=True)
    lg2 = jnp.where(iota == a1, -jnp.inf, lg)
    m2 = jnp.max(lg2, axis=1, keepdims=True)
    a2 = jnp.min(jnp.where(lg2 == m2, iota, big), axis=1, keepdims=True)
    ex = jnp.exp(lg - m1)
    p = ex / jnp.sum(ex, axis=1, keepdims=True)
    p1 = jnp.sum(jnp.where(iota == a1, p, 0.0), axis=1, keepdims=True)
    p2 = jnp.sum(jnp.where(iota == a2, p, 0.0), axis=1, keepdims=True)
    sb = jnp.broadcast_to(scale_ref[...], (T, E))
    s1 = jnp.sum(jnp.where(iota == a1, sb, 0.0), axis=1, keepdims=True)
    s2 = jnp.sum(jnp.where(iota == a2, sb, 0.0), axis=1, keepdims=True)
    rn = p1 + p2
    rn = jnp.where(rn > 0.0, rn, 1.0)
    ids_ref[...] = jnp.concatenate([a1, a2], axis=1)
    w0_ref[...] = jnp.broadcast_to(p1 / rn * s1, (T, L))
    w1_ref[...] = jnp.broadcast_to(p2 / rn * s2, (T, L))


_routing = pl.pallas_call(
    _routing_body,
    out_shape=(
        jax.ShapeDtypeStruct((T, K), jnp.int32),
        jax.ShapeDtypeStruct((T, L), jnp.float32),
        jax.ShapeDtypeStruct((T, L), jnp.float32),
    ),
)


def _dispatch_body(x_hbm, pos0_hbm, pos1_hbm, xs_hbm, buf, i0, i1):
    wid = lax.axis_index("s") * NC + lax.axis_index("c")
    base = wid * TW
    pltpu.sync_copy(pos0_hbm.at[pl.ds(base, TW)], i0)
    pltpu.sync_copy(pos1_hbm.at[pl.ds(base, TW)], i1)
    pltpu.sync_copy(x_hbm.at[pl.ds(base, TW)], buf)
    pltpu.sync_copy(buf, xs_hbm.at[i0])
    pltpu.sync_copy(buf, xs_hbm.at[i1])


_dispatch = functools.partial(
    pl.kernel,
    mesh=plsc.VectorSubcoreMesh(core_axis_name="c", subcore_axis_name="s"),
    out_type=jax.ShapeDtypeStruct((NPAD, D), jnp.float32),
    scratch_types=[
        pltpu.VMEM((TW, D), jnp.float32),
        pltpu.VMEM((TW,), jnp.int32),
        pltpu.VMEM((TW,), jnp.int32),
    ],
)(_dispatch_body)


def _ffn_body(sp_ref, xs_ref, wg_ref, wu_ref, wd_ref, out_ref):
    nb = sp_ref[G_MAX]

    @pl.when(pl.program_id(0) < nb)
    def _():
        xb = xs_ref[...].astype(jnp.bfloat16)
        g = jnp.dot(xb, wg_ref[0], preferred_element_type=jnp.float32)
        u = jnp.dot(xb, wu_ref[0], preferred_element_type=jnp.float32)
        gelu = g * 0.5 * (1.0 + lax.erf(g * 0.7071067811865476))
        act = (gelu * u).astype(jnp.bfloat16)
        out_ref[...] = jnp.dot(act, wd_ref[0], preferred_element_type=jnp.float32)


_ffn = pl.pallas_call(
    _ffn_body,
    grid_spec=pltpu.PrefetchScalarGridSpec(
        num_scalar_prefetch=1,
        grid=(G_MAX,),
        in_specs=[
            pl.BlockSpec((B, D), lambda g, sp: (g, 0)),
            pl.BlockSpec((1, D, F), lambda g, sp: (sp[g], 0, 0)),
            pl.BlockSpec((1, D, F), lambda g, sp: (sp[g], 0, 0)),
            pl.BlockSpec((1, F, D), lambda g, sp: (sp[g], 0, 0)),
        ],
        out_specs=pl.BlockSpec((B, D), lambda g, sp: (g, 0)),
    ),
    out_shape=jax.ShapeDtypeStruct((NPAD, D), jnp.float32),
)


def _combine_body(ys_hbm, pos0_hbm, pos1_hbm, w0_hbm, w1_hbm, out_hbm,
                  i0, i1, w0v, w1v, abuf, bbuf, obuf, sem):
    wid = lax.axis_index("s") * NC + lax.axis_index("c")
    for c in range(TW // CH):
        tb = wid * TW + c * CH
        pltpu.sync_copy(pos0_hbm.at[pl.ds(tb, CH)], i0)
        pltpu.sync_copy(pos1_hbm.at[pl.ds(tb, CH)], i1)
        pltpu.sync_copy(w0_hbm.at[pl.ds(tb, CH)], w0v)
        pltpu.sync_copy(w1_hbm.at[pl.ds(tb, CH)], w1v)
        pltpu.async_copy(ys_hbm.at[i0], abuf, sem).wait()
        pltpu.async_copy(ys_hbm.at[i1], bbuf, sem).wait()
        for j in range(CH):
            w0j = w0v[j]
            w1j = w1v[j]

            def _lane(ch, carry, j=j, w0j=w0j, w1j=w1j):
                sl = pl.ds(ch * L, L)
                obuf[j, sl] = w0j * abuf[j, sl] + w1j * bbuf[j, sl]
                return carry

            lax.fori_loop(0, D // L, _lane, 0)
        pltpu.sync_copy(obuf, out_hbm.at[pl.ds(tb, CH)])


_combine = functools.partial(
    pl.kernel,
    mesh=plsc.VectorSubcoreMesh(core_axis_name="c", subcore_axis_name="s"),
    out_type=jax.ShapeDtypeStruct((T, D), jnp.float32),
    scratch_types=[
        pltpu.VMEM((CH,), jnp.int32),
        pltpu.VMEM((CH,), jnp.int32),
        pltpu.VMEM((CH, L), jnp.float32),
        pltpu.VMEM((CH, L), jnp.float32),
        pltpu.VMEM((CH, D), jnp.float32),
        pltpu.VMEM((CH, D), jnp.float32),
        pltpu.VMEM((CH, D), jnp.float32),
        pltpu.SemaphoreType.DMA,
    ],
)(_combine_body)


def kernel(x, router_logits, per_expert_scale, w_gate, w_up, w_down):
    ids, w0b, w1b = _routing(router_logits, per_expert_scale.reshape(1, E))

    eflat = ids.reshape(A)
    oh = (eflat[:, None] == jnp.arange(E, dtype=jnp.int32)[None, :]).astype(jnp.int32)
    csum = jnp.cumsum(oh, axis=0)
    rank = jnp.sum((csum - oh) * oh, axis=1)
    counts = csum[-1]
    nblk = (counts + B - 1) // B
    cum = jnp.cumsum(nblk)
    pad_off = (B * (cum - nblk)).astype(jnp.int32)
    pos = (jnp.take(pad_off, eflat) + rank).astype(jnp.int32)
    pos2 = pos.reshape(T, K)
    pos0, pos1 = pos2[:, 0], pos2[:, 1]
    g_ids = jnp.arange(G_MAX, dtype=jnp.int32)
    g_eff = jnp.minimum(g_ids, cum[-1] - 1)
    be = jnp.sum((g_eff[:, None] >= cum[None, :]).astype(jnp.int32), axis=1)
    sp = jnp.concatenate([be, cum[-1:]]).astype(jnp.int32)

    xs = _dispatch(x, pos0, pos1)
    ys = _ffn(sp, xs,
              w_gate.astype(jnp.bfloat16),
              w_up.astype(jnp.bfloat16),
              w_down.astype(jnp.bfloat16))
    return _combine(ys, pos0, pos1, w0b, w1b)


# speedup vs baseline: 2.6830x; 1.0616x over previous
import functools

import jax
import jax.numpy as jnp
from jax import lax
from jax.experimental import pallas as pl
from jax.experimental.pallas import tpu as pltpu
from jax.experimental.pallas import tpu_sc as plsc

T, D, E, F, K = 2048, 1024, 8, 2048, 2
A = T * K
B = 256
G_MAX = 24
NPAD = G_MAX * B

SP_N = 32
NC, NS, L = 2, 16, 16
NW = NC * NS
TW = T // NW
CH = 32


def _cumsum_rows(x):
    n = x.shape[0]
    s = 1
    while s < n:
        shifted = jnp.concatenate(
            [jnp.zeros((s,) + x.shape[1:], x.dtype), x[:-s]], axis=0)
        x = x + shifted
        s *= 2
    return x


def _cumsum_lanes(x):
    n = x.shape[1]
    s = 1
    while s < n:
        shifted = jnp.concatenate(
            [jnp.zeros(x.shape[:1] + (s,), x.dtype), x[:, :-s]], axis=1)
        x = x + shifted
        s *= 2
    return x


def _routing_body(logits_ref, scale_ref, pos0_ref, pos1_ref, w0_ref, w1_ref,
                  sp_ref):
    lg = logits_ref[...]
    iota = lax.broadcasted_iota(jnp.int32, (T, E), 1)
    big = jnp.int32(E)
    m1 = jnp.max(lg, axis=1, keepdims=True)
    a1 = jnp.min(jnp.where(lg == m1, iota, big), axis=1, keepdims=True)
    lg2 = jnp.where(iota == a1, -jnp.inf, lg)
    m2 = jnp.max(lg2, axis=1, keepdims=True)
    a2 = jnp.min(jnp.where(lg2 == m2, iota, big), axis=1, keepdims=True)
    ex = jnp.exp(lg - m1)
    p = ex / jnp.sum(ex, axis=1, keepdims=True)
    p1 = jnp.sum(jnp.where(iota == a1, p, 0.0), axis=1, keepdims=True)
    p2 = jnp.sum(jnp.where(iota == a2, p, 0.0), axis=1, keepdims=True)
    sb = jnp.broadcast_to(scale_ref[...], (T, E))
    s1 = jnp.sum(jnp.where(iota == a1, sb, 0.0), axis=1, keepdims=True)
    s2 = jnp.sum(jnp.where(iota == a2, sb, 0.0), axis=1, keepdims=True)
    rn = p1 + p2
    rn = jnp.where(rn > 0.0, rn, 1.0)
    w0_ref[...] = jnp.broadcast_to(p1 / rn * s1, (T, L))
    w1_ref[...] = jnp.broadcast_to(p2 / rn * s2, (T, L))

    oh1 = (iota == a1).astype(jnp.int32)
    oh2 = (iota == a2).astype(jnp.int32)
    ohs = oh1 + oh2
    csi = _cumsum_rows(ohs)
    cs = csi - ohs
    counts = csi[T - 1:T, :]
    nblk = (counts + B - 1) // B
    cum = _cumsum_lanes(nblk)
    pad_off = B * (cum - nblk)
    posall = cs + pad_off
    pos0_ref[...] = jnp.sum(jnp.where(oh1 == 1, posall, 0), axis=1,
                            keepdims=True)
    pos1_ref[...] = jnp.sum(jnp.where(oh2 == 1, posall, 0), axis=1,
                            keepdims=True)

    total = cum[:, E - 1:E]
    r = lax.broadcasted_iota(jnp.int32, (SP_N, E), 0)
    g_eff = jnp.minimum(r, jnp.broadcast_to(total, (SP_N, E)) - 1)
    cmp = (g_eff >= jnp.broadcast_to(cum, (SP_N, E))).astype(jnp.int32)
    bsum = jnp.sum(cmp, axis=1, keepdims=True)
    ridx = lax.broadcasted_iota(jnp.int32, (SP_N, 1), 0)
    sp_ref[...] = jnp.where(ridx >= G_MAX, jnp.broadcast_to(total, (SP_N, 1)),
                            bsum)


_routing = pl.pallas_call(
    _routing_body,
    out_shape=(
        jax.ShapeDtypeStruct((T, 1), jnp.int32),
        jax.ShapeDtypeStruct((T, 1), jnp.int32),
        jax.ShapeDtypeStruct((T, L), jnp.float32),
        jax.ShapeDtypeStruct((T, L), jnp.float32),
        jax.ShapeDtypeStruct((SP_N, 1), jnp.int32),
    ),
)


def _dispatch_body(x_hbm, pos0_hbm, pos1_hbm, xs_hbm, buf, i0, i1):
    wid = lax.axis_index("s") * NC + lax.axis_index("c")
    base = wid * TW
    pltpu.sync_copy(pos0_hbm.at[pl.ds(base, TW)], i0)
    pltpu.sync_copy(pos1_hbm.at[pl.ds(base, TW)], i1)
    pltpu.sync_copy(x_hbm.at[pl.ds(base, TW)], buf)
    pltpu.sync_copy(buf, xs_hbm.at[i0])
    pltpu.sync_copy(buf, xs_hbm.at[i1])


_dispatch = functools.partial(
    pl.kernel,
    mesh=plsc.VectorSubcoreMesh(core_axis_name="c", subcore_axis_name="s"),
    out_type=jax.ShapeDtypeStruct((NPAD, D), jnp.float32),
    scratch_types=[
        pltpu.VMEM((TW, D), jnp.float32),
        pltpu.VMEM((TW,), jnp.int32),
        pltpu.VMEM((TW,), jnp.int32),
    ],
)(_dispatch_body)


def _ffn_body(sp_ref, xs_ref, wg_ref, wu_ref, wd_ref, out_ref):
    nb = sp_ref[G_MAX]

    @pl.when(pl.program_id(0) < nb)
    def _():
        xb = xs_ref[...].astype(jnp.bfloat16)
        g = jnp.dot(xb, wg_ref[0], preferred_element_type=jnp.float32)
        u = jnp.dot(xb, wu_ref[0], preferred_element_type=jnp.float32)
        gelu = g * 0.5 * (1.0 + lax.erf(g * 0.7071067811865476))
        act = (gelu * u).astype(jnp.bfloat16)
        out_ref[...] = jnp.dot(act, wd_ref[0], preferred_element_type=jnp.float32)


_ffn = pl.pallas_call(
    _ffn_body,
    grid_spec=pltpu.PrefetchScalarGridSpec(
        num_scalar_prefetch=1,
        grid=(G_MAX,),
        in_specs=[
            pl.BlockSpec((B, D), lambda g, sp: (g, 0)),
            pl.BlockSpec((1, D, F), lambda g, sp: (sp[g], 0, 0)),
            pl.BlockSpec((1, D, F), lambda g, sp: (sp[g], 0, 0)),
            pl.BlockSpec((1, F, D), lambda g, sp: (sp[g], 0, 0)),
        ],
        out_specs=pl.BlockSpec((B, D), lambda g, sp: (g, 0)),
    ),
    out_shape=jax.ShapeDtypeStruct((NPAD, D), jnp.float32),
)


def _combine_body(ys_hbm, pos0_hbm, pos1_hbm, w0_hbm, w1_hbm, out_hbm,
                  i0, i1, w0v, w1v, abuf, bbuf, obuf, sem):
    wid = lax.axis_index("s") * NC + lax.axis_index("c")
    for c in range(TW // CH):
        tb = wid * TW + c * CH
        pltpu.sync_copy(pos0_hbm.at[pl.ds(tb, CH)], i0)
        pltpu.sync_copy(pos1_hbm.at[pl.ds(tb, CH)], i1)
        pltpu.sync_copy(w0_hbm.at[pl.ds(tb, CH)], w0v)
        pltpu.sync_copy(w1_hbm.at[pl.ds(tb, CH)], w1v)
        pltpu.async_copy(ys_hbm.at[i0], abuf, sem).wait()
        pltpu.async_copy(ys_hbm.at[i1], bbuf, sem).wait()
        for j in range(CH):
            w0j = w0v[j]
            w1j = w1v[j]

            def _lane(ch, carry, j=j, w0j=w0j, w1j=w1j):
                sl = pl.ds(ch * L, L)
                obuf[j, sl] = w0j * abuf[j, sl] + w1j * bbuf[j, sl]
                return carry

            lax.fori_loop(0, D // L, _lane, 0)
        pltpu.sync_copy(obuf, out_hbm.at[pl.ds(tb, CH)])


_combine = functools.partial(
    pl.kernel,
    mesh=plsc.VectorSubcoreMesh(core_axis_name="c", subcore_axis_name="s"),
    out_type=jax.ShapeDtypeStruct((T, D), jnp.float32),
    scratch_types=[
        pltpu.VMEM((CH,), jnp.int32),
        pltpu.VMEM((CH,), jnp.int32),
        pltpu.VMEM((CH, L), jnp.float32),
        pltpu.VMEM((CH, L), jnp.float32),
        pltpu.VMEM((CH, D), jnp.float32),
        pltpu.VMEM((CH, D), jnp.float32),
        pltpu.VMEM((CH, D), jnp.float32),
        pltpu.SemaphoreType.DMA,
    ],
)(_combine_body)


def kernel(x, router_logits, per_expert_scale, w_gate, w_up, w_down):
    pos0, pos1, w0b, w1b, spc = _routing(router_logits,
                                         per_expert_scale.reshape(1, E))
    pos0 = pos0.reshape(T)
    pos1 = pos1.reshape(T)
    sp = spc.reshape(SP_N)[:G_MAX + 1]

    xs = _dispatch(x, pos0, pos1)
    ys = _ffn(sp, xs,
              w_gate.astype(jnp.bfloat16),
              w_up.astype(jnp.bfloat16),
              w_down.astype(jnp.bfloat16))
    return _combine(ys, pos0, pos1, w0b, w1b)


# speedup vs baseline: 3.3823x; 1.2606x over previous
import functools

import jax
import jax.numpy as jnp
from jax import lax
from jax.experimental import pallas as pl
from jax.experimental.pallas import tpu as pltpu
from jax.experimental.pallas import tpu_sc as plsc

T, D, E, F, K = 2048, 1024, 8, 2048, 2
A = T * K
B = 128
G_MAX = 39
NPAD = G_MAX * B

SP_N = 48
NC, NS, L = 2, 16, 16
NW = NC * NS
TW = T // NW
CH = 32


def _cumsum_rows(x):
    n = x.shape[0]
    s = 1
    while s < n:
        shifted = jnp.concatenate(
            [jnp.zeros((s,) + x.shape[1:], x.dtype), x[:-s]], axis=0)
        x = x + shifted
        s *= 2
    return x


def _cumsum_lanes(x):
    n = x.shape[1]
    s = 1
    while s < n:
        shifted = jnp.concatenate(
            [jnp.zeros(x.shape[:1] + (s,), x.dtype), x[:, :-s]], axis=1)
        x = x + shifted
        s *= 2
    return x


def _routing_body(logits_ref, scale_ref, pos0_ref, pos1_ref, w0_ref, w1_ref,
                  sp_ref):
    lg = logits_ref[...]
    iota = lax.broadcasted_iota(jnp.int32, (T, E), 1)
    big = jnp.int32(E)
    m1 = jnp.max(lg, axis=1, keepdims=True)
    a1 = jnp.min(jnp.where(lg == m1, iota, big), axis=1, keepdims=True)
    lg2 = jnp.where(iota == a1, -jnp.inf, lg)
    m2 = jnp.max(lg2, axis=1, keepdims=True)
    a2 = jnp.min(jnp.where(lg2 == m2, iota, big), axis=1, keepdims=True)
    ex = jnp.exp(lg - m1)
    p = ex / jnp.sum(ex, axis=1, keepdims=True)
    p1 = jnp.sum(jnp.where(iota == a1, p, 0.0), axis=1, keepdims=True)
    p2 = jnp.sum(jnp.where(iota == a2, p, 0.0), axis=1, keepdims=True)
    sb = jnp.broadcast_to(scale_ref[...], (T, E))
    s1 = jnp.sum(jnp.where(iota == a1, sb, 0.0), axis=1, keepdims=True)
    s2 = jnp.sum(jnp.where(iota == a2, sb, 0.0), axis=1, keepdims=True)
    rn = p1 + p2
    rn = jnp.where(rn > 0.0, rn, 1.0)
    w0_ref[...] = jnp.broadcast_to(p1 / rn * s1, (T, L))
    w1_ref[...] = jnp.broadcast_to(p2 / rn * s2, (T, L))

    oh1 = (iota == a1).astype(jnp.int32)
    oh2 = (iota == a2).astype(jnp.int32)
    ohs = oh1 + oh2
    csi = _cumsum_rows(ohs)
    cs = csi - ohs
    counts = csi[T - 1:T, :]
    nblk = (counts + B - 1) // B
    cum = _cumsum_lanes(nblk)
    pad_off = B * (cum - nblk)
    posall = cs + pad_off
    pos0_ref[...] = jnp.sum(jnp.where(oh1 == 1, posall, 0), axis=1,
                            keepdims=True)
    pos1_ref[...] = jnp.sum(jnp.where(oh2 == 1, posall, 0), axis=1,
                            keepdims=True)

    total = cum[:, E - 1:E]
    r = lax.broadcasted_iota(jnp.int32, (SP_N, E), 0)
    g_eff = jnp.minimum(r, jnp.broadcast_to(total, (SP_N, E)) - 1)
    cmp = (g_eff >= jnp.broadcast_to(cum, (SP_N, E))).astype(jnp.int32)
    bsum = jnp.sum(cmp, axis=1, keepdims=True)
    ridx = lax.broadcasted_iota(jnp.int32, (SP_N, 1), 0)
    sp_ref[...] = jnp.where(ridx >= G_MAX, jnp.broadcast_to(total, (SP_N, 1)),
                            bsum)


_routing = pl.pallas_call(
    _routing_body,
    out_shape=(
        jax.ShapeDtypeStruct((T, 1), jnp.int32),
        jax.ShapeDtypeStruct((T, 1), jnp.int32),
        jax.ShapeDtypeStruct((T, L), jnp.float32),
        jax.ShapeDtypeStruct((T, L), jnp.float32),
        jax.ShapeDtypeStruct((SP_N, 1), jnp.int32),
    ),
)


def _dispatch_body(x_hbm, pos0_hbm, pos1_hbm, xs_hbm, buf, i0, i1):
    wid = lax.axis_index("s") * NC + lax.axis_index("c")
    base = wid * TW
    pltpu.sync_copy(pos0_hbm.at[pl.ds(base, TW)], i0)
    pltpu.sync_copy(pos1_hbm.at[pl.ds(base, TW)], i1)
    pltpu.sync_copy(x_hbm.at[pl.ds(base, TW)], buf)
    pltpu.sync_copy(buf, xs_hbm.at[i0])
    pltpu.sync_copy(buf, xs_hbm.at[i1])


_dispatch = functools.partial(
    pl.kernel,
    mesh=plsc.VectorSubcoreMesh(core_axis_name="c", subcore_axis_name="s"),
    out_type=jax.ShapeDtypeStruct((NPAD, D), jnp.float32),
    scratch_types=[
        pltpu.VMEM((TW, D), jnp.float32),
        pltpu.VMEM((TW,), jnp.int32),
        pltpu.VMEM((TW,), jnp.int32),
    ],
)(_dispatch_body)


def _ffn_body(sp_ref, xs_ref, wg_ref, wu_ref, wd_ref, out_ref):
    nb = sp_ref[G_MAX]

    @pl.when(pl.program_id(0) < nb)
    def _():
        xb = xs_ref[...].astype(jnp.bfloat16)
        wg = wg_ref[0].astype(jnp.bfloat16)
        wu = wu_ref[0].astype(jnp.bfloat16)
        g = jnp.dot(xb, wg, preferred_element_type=jnp.float32)
        u = jnp.dot(xb, wu, preferred_element_type=jnp.float32)
        gelu = g * 0.5 * (1.0 + lax.erf(g * 0.7071067811865476))
        act = (gelu * u).astype(jnp.bfloat16)
        wd = wd_ref[0].astype(jnp.bfloat16)
        out_ref[...] = jnp.dot(act, wd, preferred_element_type=jnp.float32)


_ffn = pl.pallas_call(
    _ffn_body,
    grid_spec=pltpu.PrefetchScalarGridSpec(
        num_scalar_prefetch=1,
        grid=(G_MAX,),
        in_specs=[
            pl.BlockSpec((B, D), lambda g, sp: (g, 0)),
            pl.BlockSpec((1, D, F), lambda g, sp: (sp[g], 0, 0)),
            pl.BlockSpec((1, D, F), lambda g, sp: (sp[g], 0, 0)),
            pl.BlockSpec((1, F, D), lambda g, sp: (sp[g], 0, 0)),
        ],
        out_specs=pl.BlockSpec((B, D), lambda g, sp: (g, 0)),
    ),
    out_shape=jax.ShapeDtypeStruct((NPAD, D), jnp.float32),
)


def _combine_body(ys_hbm, pos0_hbm, pos1_hbm, w0_hbm, w1_hbm, out_hbm,
                  i0, i1, w0v, w1v, abuf, bbuf, obuf, sem):
    wid = lax.axis_index("s") * NC + lax.axis_index("c")
    for c in range(TW // CH):
        tb = wid * TW + c * CH
        pltpu.sync_copy(pos0_hbm.at[pl.ds(tb, CH)], i0)
        pltpu.sync_copy(pos1_hbm.at[pl.ds(tb, CH)], i1)
        pltpu.sync_copy(w0_hbm.at[pl.ds(tb, CH)], w0v)
        pltpu.sync_copy(w1_hbm.at[pl.ds(tb, CH)], w1v)
        pltpu.async_copy(ys_hbm.at[i0], abuf, sem).wait()
        pltpu.async_copy(ys_hbm.at[i1], bbuf, sem).wait()
        for j in range(CH):
            w0j = w0v[j]
            w1j = w1v[j]

            def _lane(ch, carry, j=j, w0j=w0j, w1j=w1j):
                sl = pl.ds(ch * L, L)
                obuf[j, sl] = w0j * abuf[j, sl] + w1j * bbuf[j, sl]
                return carry

            lax.fori_loop(0, D // L, _lane, 0)
        pltpu.sync_copy(obuf, out_hbm.at[pl.ds(tb, CH)])


_combine = functools.partial(
    pl.kernel,
    mesh=plsc.VectorSubcoreMesh(core_axis_name="c", subcore_axis_name="s"),
    out_type=jax.ShapeDtypeStruct((T, D), jnp.float32),
    scratch_types=[
        pltpu.VMEM((CH,), jnp.int32),
        pltpu.VMEM((CH,), jnp.int32),
        pltpu.VMEM((CH, L), jnp.float32),
        pltpu.VMEM((CH, L), jnp.float32),
        pltpu.VMEM((CH, D), jnp.float32),
        pltpu.VMEM((CH, D), jnp.float32),
        pltpu.VMEM((CH, D), jnp.float32),
        pltpu.SemaphoreType.DMA,
    ],
)(_combine_body)


def kernel(x, router_logits, per_expert_scale, w_gate, w_up, w_down):
    pos0, pos1, w0b, w1b, spc = _routing(router_logits,
                                         per_expert_scale.reshape(1, E))
    pos0 = pos0.reshape(T)
    pos1 = pos1.reshape(T)
    sp = spc.reshape(SP_N)[:G_MAX + 1]

    xs = _dispatch(x, pos0, pos1)
    ys = _ffn(sp, xs, w_gate, w_up, w_down)
    return _combine(ys, pos0, pos1, w0b, w1b)


# speedup vs baseline: 3.4598x; 1.0229x over previous
import functools

import jax
import jax.numpy as jnp
from jax import lax
from jax.experimental import pallas as pl
from jax.experimental.pallas import tpu as pltpu
from jax.experimental.pallas import tpu_sc as plsc

T, D, E, F, K = 2048, 1024, 8, 2048, 2
A = T * K
B = 128
G_MAX = 39
NPAD = G_MAX * B

SP_N = 48
NC, NS, L = 2, 16, 16
NW = NC * NS
TW = T // NW
CH = 32
LW = 128


def _cumsum_rows(x):
    n = x.shape[0]
    s = 1
    while s < n:
        shifted = jnp.concatenate(
            [jnp.zeros((s,) + x.shape[1:], x.dtype), x[:-s]], axis=0)
        x = x + shifted
        s *= 2
    return x


def _cumsum_lanes(x):
    n = x.shape[1]
    s = 1
    while s < n:
        shifted = jnp.concatenate(
            [jnp.zeros(x.shape[:1] + (s,), x.dtype), x[:, :-s]], axis=1)
        x = x + shifted
        s *= 2
    return x


def _routing_body(logits_ref, scale_ref, pos0_ref, pos1_ref, w0_ref, w1_ref,
                  sp_ref):
    lg = logits_ref[...]
    iota = lax.broadcasted_iota(jnp.int32, (T, E), 1)
    big = jnp.int32(E)
    m1 = jnp.max(lg, axis=1, keepdims=True)
    a1 = jnp.min(jnp.where(lg == m1, iota, big), axis=1, keepdims=True)
    lg2 = jnp.where(iota == a1, -jnp.inf, lg)
    m2 = jnp.max(lg2, axis=1, keepdims=True)
    a2 = jnp.min(jnp.where(lg2 == m2, iota, big), axis=1, keepdims=True)
    ex = jnp.exp(lg - m1)
    p = ex / jnp.sum(ex, axis=1, keepdims=True)
    p1 = jnp.sum(jnp.where(iota == a1, p, 0.0), axis=1, keepdims=True)
    p2 = jnp.sum(jnp.where(iota == a2, p, 0.0), axis=1, keepdims=True)
    sb = jnp.broadcast_to(scale_ref[...], (T, E))
    s1 = jnp.sum(jnp.where(iota == a1, sb, 0.0), axis=1, keepdims=True)
    s2 = jnp.sum(jnp.where(iota == a2, sb, 0.0), axis=1, keepdims=True)
    rn = p1 + p2
    rn = jnp.where(rn > 0.0, rn, 1.0)
    w0_ref[...] = jnp.broadcast_to(p1 / rn * s1, (T, LW))
    w1_ref[...] = jnp.broadcast_to(p2 / rn * s2, (T, LW))

    oh1 = (iota == a1).astype(jnp.int32)
    oh2 = (iota == a2).astype(jnp.int32)
    ohs = oh1 + oh2
    csi = _cumsum_rows(ohs)
    cs = csi - ohs
    counts = csi[T - 1:T, :]
    nblk = (counts + B - 1) // B
    cum = _cumsum_lanes(nblk)
    pad_off = B * (cum - nblk)
    posall = cs + pad_off
    pos0_ref[...] = jnp.sum(jnp.where(oh1 == 1, posall, 0), axis=1,
                            keepdims=True)
    pos1_ref[...] = jnp.sum(jnp.where(oh2 == 1, posall, 0), axis=1,
                            keepdims=True)

    total = cum[:, E - 1:E]
    r = lax.broadcasted_iota(jnp.int32, (SP_N, E), 0)
    g_eff = jnp.minimum(r, jnp.broadcast_to(total, (SP_N, E)) - 1)
    cmp = (g_eff >= jnp.broadcast_to(cum, (SP_N, E))).astype(jnp.int32)
    bsum = jnp.sum(cmp, axis=1, keepdims=True)
    ridx = lax.broadcasted_iota(jnp.int32, (SP_N, 1), 0)
    sp_ref[...] = jnp.where(ridx >= G_MAX, jnp.broadcast_to(total, (SP_N, 1)),
                            bsum)


_routing = pl.pallas_call(
    _routing_body,
    out_shape=(
        jax.ShapeDtypeStruct((T, 1), jnp.int32),
        jax.ShapeDtypeStruct((T, 1), jnp.int32),
        jax.ShapeDtypeStruct((T, LW), jnp.float32),
        jax.ShapeDtypeStruct((T, LW), jnp.float32),
        jax.ShapeDtypeStruct((SP_N, 1), jnp.int32),
    ),
)


def _dispatch_body(x_hbm, pos0_hbm, pos1_hbm, w0_hbm, w1_hbm,
                   xs_hbm, ws_hbm, buf, i0, i1, wb):
    wid = lax.axis_index("s") * NC + lax.axis_index("c")
    base = wid * TW
    pltpu.sync_copy(pos0_hbm.at[pl.ds(base, TW)], i0)
    pltpu.sync_copy(pos1_hbm.at[pl.ds(base, TW)], i1)
    pltpu.sync_copy(x_hbm.at[pl.ds(base, TW)], buf)
    pltpu.sync_copy(buf, xs_hbm.at[i0])
    pltpu.sync_copy(buf, xs_hbm.at[i1])
    pltpu.sync_copy(w0_hbm.at[pl.ds(base, TW)], wb)
    pltpu.sync_copy(wb, ws_hbm.at[i0])
    pltpu.sync_copy(w1_hbm.at[pl.ds(base, TW)], wb)
    pltpu.sync_copy(wb, ws_hbm.at[i1])


_dispatch = functools.partial(
    pl.kernel,
    mesh=plsc.VectorSubcoreMesh(core_axis_name="c", subcore_axis_name="s"),
    out_type=[
        jax.ShapeDtypeStruct((NPAD, D), jnp.float32),
        jax.ShapeDtypeStruct((NPAD, LW), jnp.float32),
    ],
    scratch_types=[
        pltpu.VMEM((TW, D), jnp.float32),
        pltpu.VMEM((TW,), jnp.int32),
        pltpu.VMEM((TW,), jnp.int32),
        pltpu.VMEM((TW, LW), jnp.float32),
    ],
)(_dispatch_body)


def _ffn_body(sp_ref, xs_ref, ws_ref, wg_ref, wu_ref, wd_ref, out_ref):
    nb = sp_ref[G_MAX]

    @pl.when(pl.program_id(0) < nb)
    def _():
        xb = xs_ref[...].astype(jnp.bfloat16)
        wg = wg_ref[0].astype(jnp.bfloat16)
        wu = wu_ref[0].astype(jnp.bfloat16)
        g = jnp.dot(xb, wg, preferred_element_type=jnp.float32)
        u = jnp.dot(xb, wu, preferred_element_type=jnp.float32)
        gelu = g * 0.5 * (1.0 + lax.erf(g * 0.7071067811865476))
        act = (gelu * u).astype(jnp.bfloat16)
        wd = wd_ref[0].astype(jnp.bfloat16)
        y = jnp.dot(act, wd, preferred_element_type=jnp.float32)
        out_ref[...] = y * ws_ref[:, 0:1]


_ffn = pl.pallas_call(
    _ffn_body,
    grid_spec=pltpu.PrefetchScalarGridSpec(
        num_scalar_prefetch=1,
        grid=(G_MAX,),
        in_specs=[
            pl.BlockSpec((B, D), lambda g, sp: (g, 0)),
            pl.BlockSpec((B, LW), lambda g, sp: (g, 0)),
            pl.BlockSpec((1, D, F), lambda g, sp: (sp[g], 0, 0)),
            pl.BlockSpec((1, D, F), lambda g, sp: (sp[g], 0, 0)),
            pl.BlockSpec((1, F, D), lambda g, sp: (sp[g], 0, 0)),
        ],
        out_specs=pl.BlockSpec((B, D), lambda g, sp: (g, 0)),
    ),
    out_shape=jax.ShapeDtypeStruct((NPAD, D), jnp.float32),
)


def _combine_body(ys_hbm, pos0_hbm, pos1_hbm, out_hbm,
                  i0, i1, abuf, bbuf, obuf, sem):
    wid = lax.axis_index("s") * NC + lax.axis_index("c")
    for c in range(TW // CH):
        tb = wid * TW + c * CH
        pltpu.sync_copy(pos0_hbm.at[pl.ds(tb, CH)], i0)
        pltpu.sync_copy(pos1_hbm.at[pl.ds(tb, CH)], i1)
        cp0 = pltpu.make_async_copy(ys_hbm.at[i0], abuf, sem)
        cp1 = pltpu.make_async_copy(ys_hbm.at[i1], bbuf, sem)
        cp0.start()
        cp1.start()
        cp0.wait()
        cp1.wait()
        for j in range(CH):

            def _lane(ch, carry, j=j):
                sl = pl.ds(ch * L, L)
                obuf[j, sl] = abuf[j, sl] + bbuf[j, sl]
                return carry

            lax.fori_loop(0, D // L, _lane, 0)
        pltpu.sync_copy(obuf, out_hbm.at[pl.ds(tb, CH)])


_combine = functools.partial(
    pl.kernel,
    mesh=plsc.VectorSubcoreMesh(core_axis_name="c", subcore_axis_name="s"),
    out_type=jax.ShapeDtypeStruct((T, D), jnp.float32),
    scratch_types=[
        pltpu.VMEM((CH,), jnp.int32),
        pltpu.VMEM((CH,), jnp.int32),
        pltpu.VMEM((CH, D), jnp.float32),
        pltpu.VMEM((CH, D), jnp.float32),
        pltpu.VMEM((CH, D), jnp.float32),
        pltpu.SemaphoreType.DMA,
    ],
)(_combine_body)


def kernel(x, router_logits, per_expert_scale, w_gate, w_up, w_down):
    pos0, pos1, w0b, w1b, spc = _routing(router_logits,
                                         per_expert_scale.reshape(1, E))
    pos0 = pos0.reshape(T)
    pos1 = pos1.reshape(T)
    sp = spc.reshape(SP_N)[:G_MAX + 1]

    xs, ws = _dispatch(x, pos0, pos1, w0b, w1b)
    ys = _ffn(sp, xs, ws, w_gate, w_up, w_down)
    return _combine(ys, pos0, pos1)
